# baseline (speedup 1.0000x reference)
import functools

import jax
import jax.numpy as jnp
from jax import lax
from jax.experimental import pallas as pl
from jax.experimental.pallas import tpu as pltpu
from jax.experimental.pallas import tpu_sc as plsc

NUM_EMB = 100000
SEM_DIM = 4
EMB_DIM = 64
PAD = NUM_EMB * SEM_DIM

B, L, LF = 4096, 200, 4
NSEQ = B * L
NFUT = B * LF

NC, NS, LANES = 2, 16, 16
NW = NC * NS

SEQ_PER_W = NSEQ // NW
FUT_PER_W = NFUT // NW
CHUNK = 128
SEQ_CHUNKS = SEQ_PER_W // CHUNK
FUT_CHUNKS = FUT_PER_W // CHUNK
ALL_CHUNKS = SEQ_CHUNKS + FUT_CHUNKS
VEC_PER_CHUNK = CHUNK // LANES

K = 4
SET_ROWS = K * CHUNK
NPAIRS = SEQ_CHUNKS // (2 * K)

SLAB = 3200
NSLABS = SEQ_PER_W // SLAB
SLAB_CHUNKS = SLAB // CHUNK

ET = EMB_DIM // 8
BT = B // 128
LT = L // 8


def _idx_chunk(sem_v, tt_v, idx_v, chunk_id, voff):
  for u in range(VEC_PER_CHUNK):
    off = voff + u * LANES
    s = sem_v[pl.ds(off, LANES)]
    t = tt_v[pl.ds(off, LANES)]
    tc = jnp.clip(t, 0, SEM_DIM - 1)
    sc = jnp.clip(s, 0, NUM_EMB - 1)
    idx_v[chunk_id, pl.ds(u * LANES, LANES)] = tc * NUM_EMB + sc


def _sc_body(sem_h, tt_h, semf_h, ttf_h, table_h,
             out_seq_h, out_fut_h,
             sem_v, tt_v, idx_v, rows_a, rows_b, gsem, wsem):
  wid = lax.axis_index("s") * NC + lax.axis_index("c")
  base = wid * SEQ_PER_W
  basef = wid * FUT_PER_W

  def slab_loop(sidx, carry):
    soff = base + sidx * SLAB
    pltpu.sync_copy(sem_h.at[pl.ds(soff, SLAB)], sem_v)
    pltpu.sync_copy(tt_h.at[pl.ds(soff, SLAB)], tt_v)

    def chunk_loop(c, inner):
      _idx_chunk(sem_v, tt_v, idx_v, sidx * SLAB_CHUNKS + c, c * CHUNK)
      return inner
    lax.fori_loop(0, SLAB_CHUNKS, chunk_loop, 0)
    return carry
  lax.fori_loop(0, NSLABS, slab_loop, 0)

  pltpu.sync_copy(semf_h.at[pl.ds(basef, FUT_PER_W)],
                  sem_v.at[pl.ds(0, FUT_PER_W)])
  pltpu.sync_copy(ttf_h.at[pl.ds(basef, FUT_PER_W)],
                  tt_v.at[pl.ds(0, FUT_PER_W)])
  for c in range(FUT_CHUNKS):
    _idx_chunk(sem_v, tt_v, idx_v, SEQ_CHUNKS + c, c * CHUNK)

  def fire_gathers(group, rows_set):
    for b in range(K):
      pltpu.async_copy(table_h.at[idx_v.at[group * K + b]],
                       rows_set.at[pl.ds(b * CHUNK, CHUNK)], gsem)

  def fire_seq_writes(group, rows_set):
    pltpu.async_copy(rows_set,
                     out_seq_h.at[pl.ds(base + group * SET_ROWS, SET_ROWS),
                                  pl.ds(0, EMB_DIM)],
                     wsem)

  def wait_gathers(rows_set):
    pltpu.make_async_copy(out_seq_h.at[pl.ds(0, SET_ROWS), pl.ds(0, EMB_DIM)],
                          rows_set, gsem).wait()

  def wait_writes(rows_set):
    pltpu.make_async_copy(rows_set,
                          out_seq_h.at[pl.ds(0, SET_ROWS), pl.ds(0, EMB_DIM)],
                          wsem).wait()

  fire_gathers(0, rows_a)

  def pair_loop(g2, carry):
    g_a = 2 * g2

    @pl.when(g2 > 0)
    def _():
      wait_writes(rows_b)
    fire_gathers(g_a + 1, rows_b)
    wait_gathers(rows_a)
    fire_seq_writes(g_a, rows_a)
    wait_writes(rows_a)
    fire_gathers(g_a + 2, rows_a)
    wait_gathers(rows_b)
    fire_seq_writes(g_a + 1, rows_b)
    return carry
  lax.fori_loop(0, NPAIRS, pair_loop, 0)

  wait_writes(rows_b)
  wait_gathers(rows_a)
  pltpu.async_copy(rows_a, out_fut_h.at[pl.ds(basef, FUT_PER_W)], wsem)
  wait_writes(rows_a)


def _tc_body(x_ref, m_ref, i_ref, o_ref):
  x = x_ref[...][:, 0]
  m = m_ref[...][:, :, 0]
  ident = i_ref[...]
  xm = jnp.where((m != 0)[:, :, None], x, jnp.float32(0.0))
  y = jnp.einsum("ec,blc->leb", ident, xm,
                 preferred_element_type=jnp.float32)
  o_ref[...] = y.reshape(8, ET, 1, 8, 128)


@jax.jit
def _run(sem_flat, tt_flat, msk2d, semf_flat, ttf_flat, table):
  mesh = plsc.VectorSubcoreMesh(core_axis_name="c", subcore_axis_name="s",
                                num_cores=NC, num_subcores=NS)
  sc = pl.kernel(
      _sc_body,
      out_type=[
          jax.ShapeDtypeStruct((NSEQ, 128), jnp.float32),
          jax.ShapeDtypeStruct((NFUT, EMB_DIM), jnp.float32),
      ],
      mesh=mesh,
      scratch_types=[
          pltpu.VMEM((SLAB,), jnp.int32),
          pltpu.VMEM((SLAB,), jnp.int32),
          pltpu.VMEM((ALL_CHUNKS, CHUNK), jnp.int32),
          pltpu.VMEM((SET_ROWS, EMB_DIM), jnp.float32),
          pltpu.VMEM((SET_ROWS, EMB_DIM), jnp.float32),
          pltpu.SemaphoreType.DMA,
          pltpu.SemaphoreType.DMA,
      ],
      compiler_params=pltpu.CompilerParams(use_tc_tiling_on_sc=False),
  )
  rows_seq, out_fut = sc(sem_flat, tt_flat, semf_flat, ttf_flat, table)

  x4 = rows_seq.reshape(B, LT, 8, 128)
  msk3 = msk2d.reshape(B, L, 1)
  ident = jnp.concatenate(
      [jnp.eye(EMB_DIM, dtype=jnp.float32),
       jnp.zeros((EMB_DIM, 128 - EMB_DIM), jnp.float32)], axis=1)
  out5 = pl.pallas_call(
      _tc_body,
      grid=(BT, LT),
      in_specs=[
          pl.BlockSpec((128, 1, 8, 128), lambda bt, lt: (bt, lt, 0, 0)),
          pl.BlockSpec((128, 8, 1), lambda bt, lt: (bt, lt, 0)),
          pl.BlockSpec((EMB_DIM, 128), lambda bt, lt: (0, 0)),
      ],
      out_specs=pl.BlockSpec((8, ET, 1, 8, 128), lambda bt, lt: (lt, 0, bt, 0, 0)),
      out_shape=jax.ShapeDtypeStruct((L, ET, BT, 8, 128), jnp.float32),
  )(x4, msk3, ident)
  return out5, out_fut


def kernel(sem_ids, token_type_ids, seq_mask, sem_ids_fut, token_type_ids_fut,
           table):
  sem_flat = sem_ids.reshape(-1).astype(jnp.int32)
  tt_flat = token_type_ids.reshape(-1).astype(jnp.int32)
  msk2d = seq_mask.astype(jnp.int32)
  semf_flat = sem_ids_fut.reshape(-1).astype(jnp.int32)
  ttf_flat = token_type_ids_fut.reshape(-1).astype(jnp.int32)
  out5, out_fut = _run(sem_flat, tt_flat, msk2d, semf_flat, ttf_flat,
                       table.astype(jnp.float32))
  out_seq = out5.transpose(2, 4, 0, 1, 3).reshape(B, L, EMB_DIM)
  return (out_seq, out_fut.reshape(B, LF, EMB_DIM))


# speedup vs baseline: 2.3311x; 2.3311x over previous
import functools

import jax
import jax.numpy as jnp
from jax import lax
from jax.experimental import pallas as pl
from jax.experimental.pallas import tpu as pltpu
from jax.experimental.pallas import tpu_sc as plsc

NUM_EMB = 100000
SEM_DIM = 4
EMB_DIM = 64
PAD = NUM_EMB * SEM_DIM

B, L, LF = 4096, 200, 4
NSEQ = B * L
NFUT = B * LF

NC, NS, LANES = 2, 16, 16
NW = NC * NS

SEQ_PER_W = NSEQ // NW
BPW = B // NW
FUT_PER_W = NFUT // NW

ET = EMB_DIM // 8
BT = B // 128
COLV = EMB_DIM // LANES
OUTW = 129

FCHUNK = 128
FUT_CHUNKS = FUT_PER_W // FCHUNK

SLAB = 1600
SLAB_PAD = SLAB + LANES
NSLABS = SEQ_PER_W // SLAB
SLAB_SEQS = SLAB // L
VECS_PER_SEQ = 13

NSLOT = 4


def _idx_vec(sem_v, tt_v, off, use_mask, msk_v):
  s = sem_v[pl.ds(off, LANES)]
  t = tt_v[pl.ds(off, LANES)]
  tc = jnp.clip(t, 0, SEM_DIM - 1)
  sc = jnp.clip(s, 0, NUM_EMB - 1)
  idx = tc * NUM_EMB + sc
  keep = (s >= 0) & (s < NUM_EMB)
  if use_mask:
    m = msk_v[pl.ds(off, LANES)]
    keep = keep & (m != 0)
  mf = jnp.where(keep, jnp.float32(1.0), jnp.float32(0.0))
  return idx, mf


def _sc_body(sem_h, tt_h, msk_h, semf_h, ttf_h, table_h,
             out5_h, out_fut_h,
             sem_v, tt_v, msk_v, idx_v, maskf_v, fidx_v,
             rows_all, out_all, gsems, wsems):
  wid = lax.axis_index("s") * NC + lax.axis_index("c")
  base = wid * SEQ_PER_W
  basef = wid * FUT_PER_W
  iota = lax.iota(jnp.int32, LANES)

  def slab_loop(sidx, carry):
    soff = base + sidx * SLAB
    pltpu.sync_copy(sem_h.at[pl.ds(soff, SLAB)], sem_v.at[pl.ds(0, SLAB)])
    pltpu.sync_copy(tt_h.at[pl.ds(soff, SLAB)], tt_v.at[pl.ds(0, SLAB)])
    pltpu.sync_copy(msk_h.at[pl.ds(soff, SLAB)], msk_v.at[pl.ds(0, SLAB)])

    def seq_loop(c, inner):
      bcol = sidx * SLAB_SEQS + c
      bvec = jnp.full((LANES,), bcol, jnp.int32)
      for u in range(VECS_PER_SEQ):
        idx, mf = _idx_vec(sem_v, tt_v, c * L + u * LANES, True, msk_v)
        lrow = iota + (u * LANES)
        if u < VECS_PER_SEQ - 1:
          plsc.store_scatter(idx_v, [lrow, bvec], idx)
          plsc.store_scatter(maskf_v, [lrow, bvec], mf)
        else:
          tail = iota < (L - (VECS_PER_SEQ - 1) * LANES)
          plsc.store_scatter(idx_v, [lrow, bvec], idx, mask=tail)
          plsc.store_scatter(maskf_v, [lrow, bvec], mf, mask=tail)
      return inner
    lax.fori_loop(0, SLAB_SEQS, seq_loop, 0)
    return carry
  lax.fori_loop(0, NSLABS, slab_loop, 0)

  pltpu.sync_copy(semf_h.at[pl.ds(basef, FUT_PER_W)],
                  sem_v.at[pl.ds(0, FUT_PER_W)])
  pltpu.sync_copy(ttf_h.at[pl.ds(basef, FUT_PER_W)],
                  tt_v.at[pl.ds(0, FUT_PER_W)])
  for c in range(FUT_CHUNKS):
    for u in range(FCHUNK // LANES):
      idx, _ = _idx_vec(sem_v, tt_v, c * FCHUNK + u * LANES, False, None)
      fidx_v[c, pl.ds(u * LANES, LANES)] = idx

  def fire_gather(l, rows, gsem):
    pltpu.async_copy(table_h.at[idx_v.at[l]], rows, gsem)

  def wait_gather(rows, gsem):
    pltpu.make_async_copy(table_h.at[pl.ds(0, BPW)], rows, gsem).wait()

  def fire_writes(l, out_l, wsem):
    for et in range(ET):
      pltpu.async_copy(out_l.at[pl.ds(et * 8, 8), pl.ds(0, 128)],
                       out5_h.at[l, et, wid], wsem)

  def wait_writes(out_l, wsem):
    for et in range(ET):
      pltpu.make_async_copy(out_l.at[pl.ds(et * 8, 8), pl.ds(0, 128)],
                            out5_h.at[0, et, 0], wsem).wait()

  def transpose_mask(l, rows, out_l):
    for k in range(BPW // LANES):
      mv = maskf_v[l, pl.ds(k * LANES, LANES)]
      for i in range(LANES):
        bcol = k * LANES + i
        m = mv[i]
        bvec = jnp.full((LANES,), bcol, jnp.int32)
        for e0 in range(COLV):
          v = rows[bcol, pl.ds(e0 * LANES, LANES)]
          plsc.store_scatter(out_l, [iota + (e0 * LANES), bvec], v * m)

  for s in range(NSLOT):
    fire_gather(s, rows_all.at[s], gsems.at[s])

  def l_loop(l, carry):
    s = lax.rem(l, NSLOT)
    rows = rows_all.at[s]
    out_l = out_all.at[s]
    gsem = gsems.at[s]
    wsem = wsems.at[s]
    wait_gather(rows, gsem)

    @pl.when(l >= NSLOT)
    def _():
      wait_writes(out_l, wsem)
    transpose_mask(l, rows, out_l)

    @pl.when(l + NSLOT < L)
    def _():
      fire_gather(l + NSLOT, rows, gsem)
    fire_writes(l, out_l, wsem)
    return carry
  lax.fori_loop(0, L, l_loop, 0)

  for s in range(NSLOT):
    wait_writes(out_all.at[s], wsems.at[s])

  def fut_gather(c, rows, gsem):
    pltpu.async_copy(table_h.at[fidx_v.at[c]], rows, gsem)

  def fut_wait_gather(rows, gsem):
    pltpu.make_async_copy(table_h.at[pl.ds(0, FCHUNK)], rows, gsem).wait()

  def fut_write(c, rows, wsem):
    pltpu.async_copy(rows, out_fut_h.at[pl.ds(basef + c * FCHUNK, FCHUNK)],
                     wsem)

  def fut_wait_write(rows, wsem):
    pltpu.make_async_copy(rows, out_fut_h.at[pl.ds(0, FCHUNK)], wsem).wait()

  for c in range(FUT_CHUNKS):
    fut_gather(c, rows_all.at[c], gsems.at[c])
  for c in range(FUT_CHUNKS):
    fut_wait_gather(rows_all.at[c], gsems.at[c])
    fut_write(c, rows_all.at[c], wsems.at[c])
  for c in range(FUT_CHUNKS):
    fut_wait_write(rows_all.at[c], wsems.at[c])


@jax.jit
def _run(sem_flat, tt_flat, msk_flat, semf_flat, ttf_flat, table):
  mesh = plsc.VectorSubcoreMesh(core_axis_name="c", subcore_axis_name="s",
                                num_cores=NC, num_subcores=NS)
  f = pl.kernel(
      _sc_body,
      out_type=[
          jax.ShapeDtypeStruct((L, ET, BT, 8, 128), jnp.float32),
          jax.ShapeDtypeStruct((NFUT, EMB_DIM), jnp.float32),
      ],
      mesh=mesh,
      scratch_types=[
          pltpu.VMEM((SLAB_PAD,), jnp.int32),
          pltpu.VMEM((SLAB_PAD,), jnp.int32),
          pltpu.VMEM((SLAB_PAD,), jnp.int32),
          pltpu.VMEM((L, BPW), jnp.int32),
          pltpu.VMEM((L, BPW), jnp.float32),
          pltpu.VMEM((FUT_CHUNKS, FCHUNK), jnp.int32),
          pltpu.VMEM((NSLOT, BPW, EMB_DIM), jnp.float32),
          pltpu.VMEM((NSLOT, EMB_DIM, OUTW), jnp.float32),
          pltpu.SemaphoreType.DMA((NSLOT,)),
          pltpu.SemaphoreType.DMA((NSLOT,)),
      ],
      compiler_params=pltpu.CompilerParams(use_tc_tiling_on_sc=False,
                                           needs_layout_passes=False),
  )
  return f(sem_flat, tt_flat, msk_flat, semf_flat, ttf_flat, table)


def kernel(sem_ids, token_type_ids, seq_mask, sem_ids_fut, token_type_ids_fut,
           table):
  sem_flat = sem_ids.reshape(-1).astype(jnp.int32)
  tt_flat = token_type_ids.reshape(-1).astype(jnp.int32)
  msk_flat = seq_mask.reshape(-1).astype(jnp.int32)
  semf_flat = sem_ids_fut.reshape(-1).astype(jnp.int32)
  ttf_flat = token_type_ids_fut.reshape(-1).astype(jnp.int32)
  out5, out_fut = _run(sem_flat, tt_flat, msk_flat, semf_flat, ttf_flat,
                       table.astype(jnp.float32))
  out_seq = out5.transpose(2, 4, 0, 1, 3).reshape(B, L, EMB_DIM)
  return (out_seq, out_fut.reshape(B, LF, EMB_DIM))


# speedup vs baseline: 2.3941x; 1.0271x over previous
import functools

import jax
import jax.numpy as jnp
from jax import lax
from jax.experimental import pallas as pl
from jax.experimental.pallas import tpu as pltpu
from jax.experimental.pallas import tpu_sc as plsc

NUM_EMB = 100000
SEM_DIM = 4
EMB_DIM = 64
PAD = NUM_EMB * SEM_DIM

B, L, LF = 4096, 200, 4
NSEQ = B * L
NFUT = B * LF

NC, NS, LANES = 2, 16, 16
NW = NC * NS

SEQ_PER_W = NSEQ // NW
BPW = B // NW
FUT_PER_W = NFUT // NW

ET = EMB_DIM // 8
BT = B // 128
COLV = EMB_DIM // LANES
OUTW = 129

FCHUNK = 128
FUT_CHUNKS = FUT_PER_W // FCHUNK

SLAB = 1600
SLAB_PAD = SLAB + LANES
NSLABS = SEQ_PER_W // SLAB
SLAB_SEQS = SLAB // L
VECS_PER_SEQ = 13

NSLOT = 4


def _idx_vec(sem_v, tt_v, off, use_mask, msk_v):
  s = sem_v[pl.ds(off, LANES)]
  t = tt_v[pl.ds(off, LANES)]
  tc = jnp.clip(t, 0, SEM_DIM - 1)
  sc = jnp.clip(s, 0, NUM_EMB - 1)
  idx = tc * NUM_EMB + sc
  keep = (s >= 0) & (s < NUM_EMB)
  if use_mask:
    m = msk_v[pl.ds(off, LANES)]
    keep = keep & (m != 0)
  mf = jnp.where(keep, jnp.float32(1.0), jnp.float32(0.0))
  return idx, mf


def _sc_body(sem_h, tt_h, msk_h, semf_h, ttf_h, table_h,
             out5_h, out_fut_h,
             sem_v, tt_v, msk_v, idx_v, maskf_v, fidx_v,
             rows_all, out_all, gsems, wsems):
  wid = lax.axis_index("s") * NC + lax.axis_index("c")
  base = wid * SEQ_PER_W
  basef = wid * FUT_PER_W
  iota = lax.iota(jnp.int32, LANES)

  def slab_loop(sidx, carry):
    soff = base + sidx * SLAB
    pltpu.sync_copy(sem_h.at[pl.ds(soff, SLAB)], sem_v.at[pl.ds(0, SLAB)])
    pltpu.sync_copy(tt_h.at[pl.ds(soff, SLAB)], tt_v.at[pl.ds(0, SLAB)])
    pltpu.sync_copy(msk_h.at[pl.ds(soff, SLAB)], msk_v.at[pl.ds(0, SLAB)])

    def seq_loop(c, inner):
      bcol = sidx * SLAB_SEQS + c
      bvec = jnp.full((LANES,), bcol, jnp.int32)
      for u in range(VECS_PER_SEQ):
        idx, mf = _idx_vec(sem_v, tt_v, c * L + u * LANES, True, msk_v)
        lrow = iota + (u * LANES)
        if u < VECS_PER_SEQ - 1:
          plsc.store_scatter(idx_v, [lrow, bvec], idx)
          plsc.store_scatter(maskf_v, [lrow, bvec], mf)
        else:
          tail = iota < (L - (VECS_PER_SEQ - 1) * LANES)
          plsc.store_scatter(idx_v, [lrow, bvec], idx, mask=tail)
          plsc.store_scatter(maskf_v, [lrow, bvec], mf, mask=tail)
      return inner
    lax.fori_loop(0, SLAB_SEQS, seq_loop, 0)
    return carry
  lax.fori_loop(0, NSLABS, slab_loop, 0)

  pltpu.sync_copy(semf_h.at[pl.ds(basef, FUT_PER_W)],
                  sem_v.at[pl.ds(0, FUT_PER_W)])
  pltpu.sync_copy(ttf_h.at[pl.ds(basef, FUT_PER_W)],
                  tt_v.at[pl.ds(0, FUT_PER_W)])
  for c in range(FUT_CHUNKS):
    for u in range(FCHUNK // LANES):
      idx, _ = _idx_vec(sem_v, tt_v, c * FCHUNK + u * LANES, False, None)
      fidx_v[c, pl.ds(u * LANES, LANES)] = idx

  def fire_gather(l, rows, gsem):
    pltpu.async_copy(table_h.at[idx_v.at[l]], rows, gsem)

  def wait_gather(rows, gsem):
    pltpu.make_async_copy(table_h.at[pl.ds(0, BPW)], rows, gsem).wait()

  def fire_writes(l, out_l, wsem):
    pltpu.async_copy(out_l.at[pl.ds(0, ET), pl.ds(0, 8), pl.ds(0, 128)],
                     out5_h.at[l, pl.ds(0, ET), wid], wsem)

  def wait_writes(out_l, wsem):
    pltpu.make_async_copy(out_l.at[pl.ds(0, ET), pl.ds(0, 8), pl.ds(0, 128)],
                          out5_h.at[0, pl.ds(0, ET), 0], wsem).wait()

  def transpose_mask(l, rows, out_l):
    for k in range(BPW // LANES):
      mv = maskf_v[l, pl.ds(k * LANES, LANES)]
      for i in range(LANES):
        bcol = k * LANES + i
        m = mv[i]
        bvec = jnp.full((LANES,), bcol, jnp.int32)
        for e0 in range(COLV):
          e = iota + (e0 * LANES)
          v = rows[bcol, pl.ds(e0 * LANES, LANES)]
          plsc.store_scatter(out_l, [e // 8, lax.rem(e, 8), bvec], v * m)

  for s in range(NSLOT):
    fire_gather(s, rows_all.at[s], gsems.at[s])

  def l_loop(l, carry):
    s = lax.rem(l, NSLOT)
    rows = rows_all.at[s]
    out_l = out_all.at[s]
    gsem = gsems.at[s]
    wsem = wsems.at[s]
    wait_gather(rows, gsem)

    @pl.when(l >= NSLOT)
    def _():
      wait_writes(out_l, wsem)
    transpose_mask(l, rows, out_l)

    @pl.when(l + NSLOT < L)
    def _():
      fire_gather(l + NSLOT, rows, gsem)
    fire_writes(l, out_l, wsem)
    return carry
  lax.fori_loop(0, L, l_loop, 0)

  for s in range(NSLOT):
    wait_writes(out_all.at[s], wsems.at[s])

  def fut_gather(c, rows, gsem):
    pltpu.async_copy(table_h.at[fidx_v.at[c]], rows, gsem)

  def fut_wait_gather(rows, gsem):
    pltpu.make_async_copy(table_h.at[pl.ds(0, FCHUNK)], rows, gsem).wait()

  def fut_write(c, rows, wsem):
    pltpu.async_copy(rows, out_fut_h.at[pl.ds(basef + c * FCHUNK, FCHUNK)],
                     wsem)

  def fut_wait_write(rows, wsem):
    pltpu.make_async_copy(rows, out_fut_h.at[pl.ds(0, FCHUNK)], wsem).wait()

  for c in range(FUT_CHUNKS):
    fut_gather(c, rows_all.at[c], gsems.at[c])
  for c in range(FUT_CHUNKS):
    fut_wait_gather(rows_all.at[c], gsems.at[c])
    fut_write(c, rows_all.at[c], wsems.at[c])
  for c in range(FUT_CHUNKS):
    fut_wait_write(rows_all.at[c], wsems.at[c])


@jax.jit
def _run(sem_flat, tt_flat, msk_flat, semf_flat, ttf_flat, table):
  mesh = plsc.VectorSubcoreMesh(core_axis_name="c", subcore_axis_name="s",
                                num_cores=NC, num_subcores=NS)
  f = pl.kernel(
      _sc_body,
      out_type=[
          jax.ShapeDtypeStruct((L, ET, BT, 8, 128), jnp.float32),
          jax.ShapeDtypeStruct((NFUT, EMB_DIM), jnp.float32),
      ],
      mesh=mesh,
      scratch_types=[
          pltpu.VMEM((SLAB_PAD,), jnp.int32),
          pltpu.VMEM((SLAB_PAD,), jnp.int32),
          pltpu.VMEM((SLAB_PAD,), jnp.int32),
          pltpu.VMEM((L, BPW), jnp.int32),
          pltpu.VMEM((L, BPW), jnp.float32),
          pltpu.VMEM((FUT_CHUNKS, FCHUNK), jnp.int32),
          pltpu.VMEM((NSLOT, BPW, EMB_DIM), jnp.float32),
          pltpu.VMEM((NSLOT, ET, 8, OUTW), jnp.float32),
          pltpu.SemaphoreType.DMA((NSLOT,)),
          pltpu.SemaphoreType.DMA((NSLOT,)),
      ],
      compiler_params=pltpu.CompilerParams(use_tc_tiling_on_sc=False,
                                           needs_layout_passes=False),
  )
  return f(sem_flat, tt_flat, msk_flat, semf_flat, ttf_flat, table)


def kernel(sem_ids, token_type_ids, seq_mask, sem_ids_fut, token_type_ids_fut,
           table):
  sem_flat = sem_ids.reshape(-1).astype(jnp.int32)
  tt_flat = token_type_ids.reshape(-1).astype(jnp.int32)
  msk_flat = seq_mask.reshape(-1).astype(jnp.int32)
  semf_flat = sem_ids_fut.reshape(-1).astype(jnp.int32)
  ttf_flat = token_type_ids_fut.reshape(-1).astype(jnp.int32)
  out5, out_fut = _run(sem_flat, tt_flat, msk_flat, semf_flat, ttf_flat,
                       table.astype(jnp.float32))
  out_seq = out5.transpose(2, 4, 0, 1, 3).reshape(B, L, EMB_DIM)
  return (out_seq, out_fut.reshape(B, LF, EMB_DIM))
